# baseline (speedup 1.0000x reference)
import functools

import jax
import jax.numpy as jnp
from jax import lax
from jax.experimental import pallas as pl
from jax.experimental.pallas import tpu as pltpu
from jax.experimental.pallas import tpu_sc as plsc


def _table_body(e_ref, wt_ref, b_ref, o_ref):
    o_ref[...] = (
        jnp.dot(e_ref[...], wt_ref[...], preferred_element_type=jnp.float32)
        + b_ref[...]
    )


def _build_table(embed, w_t, b_row):
    v, _ = embed.shape
    d = w_t.shape[1]
    return pl.pallas_call(
        _table_body,
        out_shape=jax.ShapeDtypeStruct((v, d), jnp.float32),
    )(embed, w_t, b_row)


@functools.cache
def _make_gather(bsz, seq, vocab, dpad):
    info = plsc.get_sparse_core_info()
    nc, ns = info.num_cores, info.num_subcores
    nw = nc * ns
    assert bsz % nw == 0
    bat_per_w = bsz // nw
    tok_per_w = bat_per_w * seq
    seq_pad = (seq + 15) // 16 * 16
    n_vecs = seq_pad // 16
    seq_gather = (seq + 7) // 8 * 8
    mesh = plsc.VectorSubcoreMesh(core_axis_name="c", subcore_axis_name="s")

    @functools.partial(
        pl.kernel,
        mesh=mesh,
        compiler_params=pltpu.CompilerParams(use_tc_tiling_on_sc=False),
        out_type=jax.ShapeDtypeStruct((bsz, seq, vocab), jnp.float32),
        scratch_types=[
            pltpu.VMEM((bat_per_w, seq_pad), jnp.int32),
            pltpu.VMEM((24, dpad), jnp.float32),
            pltpu.VMEM((32, dpad), jnp.float32),
            pltpu.VMEM_SHARED((vocab, dpad), jnp.float32),
            pltpu.SemaphoreType.DMA,
            pltpu.SemaphoreType.DMA,
        ],
    )
    def gather(table_hbm, idx_hbm, out_hbm, idx2d, buf_a, buf_b,
               table_sh, sem_a, sem_b):
        cid = lax.axis_index("c")
        sid = lax.axis_index("s")
        wid = sid * nc + cid
        base_b = wid * bat_per_w

        @pl.when(sid == 0)
        def _():
            pltpu.sync_copy(table_hbm, table_sh)

        pltpu.sync_copy(idx_hbm.at[pl.ds(base_b, bat_per_w)], idx2d)
        plsc.subcore_barrier()

        h1 = seq - 24

        def fire_a(c):
            pltpu.async_copy(table_sh.at[idx2d.at[c, pl.ds(0, 24)]], buf_a, sem_a)

        def fire_b(c):
            pltpu.async_copy(table_sh.at[idx2d.at[c, pl.ds(24, 32)]], buf_b, sem_b)

        def drain_a(c):
            pltpu.make_async_copy(
                table_sh.at[idx2d.at[c, pl.ds(0, 24)]], buf_a, sem_a).wait()

        def drain_b(c):
            pltpu.make_async_copy(
                table_sh.at[idx2d.at[c, pl.ds(24, 32)]], buf_b, sem_b).wait()

        fire_a(0)
        fire_b(0)

        def body(c, carry):
            drain_a(c)
            pltpu.sync_copy(buf_a.at[:, pl.ds(0, vocab)],
                            out_hbm.at[base_b + c, pl.ds(0, 24)])

            @pl.when(c + 1 < bat_per_w)
            def _():
                fire_a(c + 1)

            drain_b(c)
            pltpu.sync_copy(buf_b.at[pl.ds(0, h1), pl.ds(0, vocab)],
                            out_hbm.at[base_b + c, pl.ds(24, h1)])

            @pl.when(c + 1 < bat_per_w)
            def _():
                fire_b(c + 1)

            return carry

        lax.fori_loop(0, bat_per_w, body, 0)

    return gather


def _head_body(ids_ref, emb_ref, wt_ref, b_ref, o_ref):
    ids = ids_ref[0, 0, :]
    seq = ids.shape[0]
    vpad = emb_ref.shape[0]
    iota = jax.lax.broadcasted_iota(jnp.int32, (seq, vpad), 1)
    onehot = (iota == ids[:, None]).astype(jnp.float32)
    h = jnp.dot(onehot, emb_ref[...], preferred_element_type=jnp.float32)
    o_ref[0] = jnp.dot(h, wt_ref[...], preferred_element_type=jnp.float32) + b_ref[...]


def _head_tc(ids3, embed_pad, w_t, b_row):
    bsz, _, seq = ids3.shape
    vocab = w_t.shape[1]
    vpad = embed_pad.shape[0]
    e = embed_pad.shape[1]
    return pl.pallas_call(
        _head_body,
        grid=(bsz,),
        in_specs=[
            pl.BlockSpec((1, 1, seq), lambda i: (i, 0, 0)),
            pl.BlockSpec((vpad, e), lambda i: (0, 0)),
            pl.BlockSpec((e, vocab), lambda i: (0, 0)),
            pl.BlockSpec((1, vocab), lambda i: (0, 0)),
        ],
        out_specs=pl.BlockSpec((1, seq, vocab), lambda i: (i, 0, 0)),
        out_shape=jax.ShapeDtypeStruct((bsz, seq, vocab), jnp.float32),
    )(ids3, embed_pad, w_t, b_row)


def kernel(input_ids, embed, W, b):
    bsz, seq = input_ids.shape
    vocab = W.shape[0]
    vpad = (vocab + 127) // 128 * 128
    ids3 = input_ids.astype(jnp.int32).reshape(bsz, 1, seq)
    embed_pad = jnp.pad(embed, ((0, vpad - vocab), (0, 0)))
    return _head_tc(ids3, embed_pad, W.T, b.reshape(1, vocab))


# speedup vs baseline: 2.2511x; 2.2511x over previous
import functools

import jax
import jax.numpy as jnp
from jax import lax
from jax.experimental import pallas as pl
from jax.experimental.pallas import tpu as pltpu
from jax.experimental.pallas import tpu_sc as plsc


def _table_body(e_ref, wt_ref, b_ref, o_ref):
    o_ref[...] = (
        jnp.dot(e_ref[...], wt_ref[...], preferred_element_type=jnp.float32)
        + b_ref[...]
    )


def _build_table(embed, w_t, b_row):
    v, _ = embed.shape
    d = w_t.shape[1]
    return pl.pallas_call(
        _table_body,
        out_shape=jax.ShapeDtypeStruct((v, d), jnp.float32),
    )(embed, w_t, b_row)


@functools.cache
def _make_gather(bsz, seq, vocab, dpad):
    info = plsc.get_sparse_core_info()
    nc, ns = info.num_cores, info.num_subcores
    nw = nc * ns
    assert bsz % nw == 0
    bat_per_w = bsz // nw
    tok_per_w = bat_per_w * seq
    seq_pad = (seq + 15) // 16 * 16
    n_vecs = seq_pad // 16
    seq_gather = (seq + 7) // 8 * 8
    mesh = plsc.VectorSubcoreMesh(core_axis_name="c", subcore_axis_name="s")

    @functools.partial(
        pl.kernel,
        mesh=mesh,
        compiler_params=pltpu.CompilerParams(use_tc_tiling_on_sc=False),
        out_type=jax.ShapeDtypeStruct((bsz, seq, vocab), jnp.float32),
        scratch_types=[
            pltpu.VMEM((bat_per_w, seq_pad), jnp.int32),
            pltpu.VMEM((24, dpad), jnp.float32),
            pltpu.VMEM((32, dpad), jnp.float32),
            pltpu.VMEM_SHARED((vocab, dpad), jnp.float32),
            pltpu.SemaphoreType.DMA,
            pltpu.SemaphoreType.DMA,
        ],
    )
    def gather(table_hbm, idx_hbm, out_hbm, idx2d, buf_a, buf_b,
               table_sh, sem_a, sem_b):
        cid = lax.axis_index("c")
        sid = lax.axis_index("s")
        wid = sid * nc + cid
        base_b = wid * bat_per_w

        @pl.when(sid == 0)
        def _():
            pltpu.sync_copy(table_hbm, table_sh)

        pltpu.sync_copy(idx_hbm.at[pl.ds(base_b, bat_per_w)], idx2d)
        plsc.subcore_barrier()

        h1 = seq - 24

        def fire_a(c):
            pltpu.async_copy(table_sh.at[idx2d.at[c, pl.ds(0, 24)]], buf_a, sem_a)

        def fire_b(c):
            pltpu.async_copy(table_sh.at[idx2d.at[c, pl.ds(24, 32)]], buf_b, sem_b)

        def drain_a(c):
            pltpu.make_async_copy(
                table_sh.at[idx2d.at[c, pl.ds(0, 24)]], buf_a, sem_a).wait()

        def drain_b(c):
            pltpu.make_async_copy(
                table_sh.at[idx2d.at[c, pl.ds(24, 32)]], buf_b, sem_b).wait()

        fire_a(0)
        fire_b(0)

        def body(c, carry):
            drain_a(c)
            pltpu.sync_copy(buf_a.at[:, pl.ds(0, vocab)],
                            out_hbm.at[base_b + c, pl.ds(0, 24)])

            @pl.when(c + 1 < bat_per_w)
            def _():
                fire_a(c + 1)

            drain_b(c)
            pltpu.sync_copy(buf_b.at[pl.ds(0, h1), pl.ds(0, vocab)],
                            out_hbm.at[base_b + c, pl.ds(24, h1)])

            @pl.when(c + 1 < bat_per_w)
            def _():
                fire_b(c + 1)

            return carry

        lax.fori_loop(0, bat_per_w, body, 0)

    return gather


_BB = 8


def _head_body(ids_ref, emb_ref, wt_ref, b_ref, o_ref):
    toks = ids_ref.shape[2]
    seq = toks // _BB
    ids = ids_ref[0, 0, :]
    vpad = emb_ref.shape[0]
    iota = jax.lax.broadcasted_iota(jnp.int32, (toks, vpad), 1)
    onehot = (iota == ids[:, None]).astype(jnp.float32)
    h = jnp.dot(onehot, emb_ref[...], preferred_element_type=jnp.float32)
    logits = jnp.dot(h, wt_ref[...], preferred_element_type=jnp.float32) + b_ref[...]
    for k in range(_BB):
        o_ref[k] = jax.lax.slice_in_dim(logits, k * seq, (k + 1) * seq, axis=0)


def _head_tc(ids3, embed_pad, w_t, b_row):
    nblk, _, toks = ids3.shape
    seq = toks // _BB
    bsz = nblk * _BB
    vocab = w_t.shape[1]
    vpad = embed_pad.shape[0]
    e = embed_pad.shape[1]
    return pl.pallas_call(
        _head_body,
        grid=(nblk,),
        in_specs=[
            pl.BlockSpec((1, 1, toks), lambda i: (i, 0, 0)),
            pl.BlockSpec((vpad, e), lambda i: (0, 0)),
            pl.BlockSpec((e, vocab), lambda i: (0, 0)),
            pl.BlockSpec((1, vocab), lambda i: (0, 0)),
        ],
        out_specs=pl.BlockSpec((_BB, seq, vocab), lambda i: (i, 0, 0)),
        out_shape=jax.ShapeDtypeStruct((bsz, seq, vocab), jnp.float32),
    )(ids3, embed_pad, w_t, b_row)


def kernel(input_ids, embed, W, b):
    bsz, seq = input_ids.shape
    vocab = W.shape[0]
    vpad = (vocab + 127) // 128 * 128
    ids3 = input_ids.astype(jnp.int32).reshape(bsz // _BB, 1, seq * _BB)
    embed_pad = jnp.pad(embed, ((0, vpad - vocab), (0, 0)))
    return _head_tc(ids3, embed_pad, W.T, b.reshape(1, vocab))


# speedup vs baseline: 10.0198x; 4.4510x over previous
import functools

import jax
import jax.numpy as jnp
from jax import lax
from jax.experimental import pallas as pl
from jax.experimental.pallas import tpu as pltpu
from jax.experimental.pallas import tpu_sc as plsc


def _table_body(e_ref, wt_ref, b_ref, o_ref):
    o_ref[...] = (
        jnp.dot(e_ref[...], wt_ref[...], preferred_element_type=jnp.float32)
        + b_ref[...]
    )


def _build_table(embed, w_t, b_row):
    v, _ = embed.shape
    d = w_t.shape[1]
    return pl.pallas_call(
        _table_body,
        out_shape=jax.ShapeDtypeStruct((v, d), jnp.float32),
    )(embed, w_t, b_row)


@functools.cache
def _make_gather(bsz, seq, vocab, dpad):
    info = plsc.get_sparse_core_info()
    nc, ns = info.num_cores, info.num_subcores
    nw = nc * ns
    assert bsz % nw == 0
    bat_per_w = bsz // nw
    tok_per_w = bat_per_w * seq
    seq_pad = (seq + 15) // 16 * 16
    n_vecs = seq_pad // 16
    seq_gather = (seq + 7) // 8 * 8
    mesh = plsc.VectorSubcoreMesh(core_axis_name="c", subcore_axis_name="s")

    @functools.partial(
        pl.kernel,
        mesh=mesh,
        compiler_params=pltpu.CompilerParams(use_tc_tiling_on_sc=False),
        out_type=jax.ShapeDtypeStruct((bsz, seq, vocab), jnp.float32),
        scratch_types=[
            pltpu.VMEM((bat_per_w, seq_pad), jnp.int32),
            pltpu.VMEM((24, dpad), jnp.float32),
            pltpu.VMEM((32, dpad), jnp.float32),
            pltpu.VMEM_SHARED((vocab, dpad), jnp.float32),
            pltpu.SemaphoreType.DMA,
            pltpu.SemaphoreType.DMA,
        ],
    )
    def gather(table_hbm, idx_hbm, out_hbm, idx2d, buf_a, buf_b,
               table_sh, sem_a, sem_b):
        cid = lax.axis_index("c")
        sid = lax.axis_index("s")
        wid = sid * nc + cid
        base_b = wid * bat_per_w

        @pl.when(sid == 0)
        def _():
            pltpu.sync_copy(table_hbm, table_sh)

        pltpu.sync_copy(idx_hbm.at[pl.ds(base_b, bat_per_w)], idx2d)
        plsc.subcore_barrier()

        h1 = seq - 24

        def fire_a(c):
            pltpu.async_copy(table_sh.at[idx2d.at[c, pl.ds(0, 24)]], buf_a, sem_a)

        def fire_b(c):
            pltpu.async_copy(table_sh.at[idx2d.at[c, pl.ds(24, 32)]], buf_b, sem_b)

        def drain_a(c):
            pltpu.make_async_copy(
                table_sh.at[idx2d.at[c, pl.ds(0, 24)]], buf_a, sem_a).wait()

        def drain_b(c):
            pltpu.make_async_copy(
                table_sh.at[idx2d.at[c, pl.ds(24, 32)]], buf_b, sem_b).wait()

        fire_a(0)
        fire_b(0)

        def body(c, carry):
            drain_a(c)
            pltpu.sync_copy(buf_a.at[:, pl.ds(0, vocab)],
                            out_hbm.at[base_b + c, pl.ds(0, 24)])

            @pl.when(c + 1 < bat_per_w)
            def _():
                fire_a(c + 1)

            drain_b(c)
            pltpu.sync_copy(buf_b.at[pl.ds(0, h1), pl.ds(0, vocab)],
                            out_hbm.at[base_b + c, pl.ds(24, h1)])

            @pl.when(c + 1 < bat_per_w)
            def _():
                fire_b(c + 1)

            return carry

        lax.fori_loop(0, bat_per_w, body, 0)

    return gather


def _head_body(idsT_ref, embT_ref, w_ref, bcol_ref, o_ref):
    vpad = embT_ref.shape[1]
    bsz = idsT_ref.shape[2]
    ids = idsT_ref[0, 0, :]
    iota = jax.lax.broadcasted_iota(jnp.int32, (vpad, bsz), 0)
    onehot_t = (iota == ids[None, :]).astype(jnp.float32)
    h_t = jnp.dot(embT_ref[...], onehot_t, preferred_element_type=jnp.float32)
    o_ref[0] = (
        jnp.dot(w_ref[...], h_t, preferred_element_type=jnp.float32)
        + bcol_ref[...]
    )


def _head_tc(ids_t3, emb_t_pad, w, b_col):
    seq, _, bsz = ids_t3.shape
    vocab, e = w.shape
    vpad = emb_t_pad.shape[1]
    return pl.pallas_call(
        _head_body,
        grid=(seq,),
        in_specs=[
            pl.BlockSpec((1, 1, bsz), lambda i: (i, 0, 0)),
            pl.BlockSpec((e, vpad), lambda i: (0, 0)),
            pl.BlockSpec((vocab, e), lambda i: (0, 0)),
            pl.BlockSpec((vocab, 1), lambda i: (0, 0)),
        ],
        out_specs=pl.BlockSpec((1, vocab, bsz), lambda i: (i, 0, 0)),
        out_shape=jax.ShapeDtypeStruct((seq, vocab, bsz), jnp.float32),
    )(ids_t3, emb_t_pad, w, b_col)


def kernel(input_ids, embed, W, b):
    bsz, seq = input_ids.shape
    vocab = W.shape[0]
    vpad = (vocab + 127) // 128 * 128
    ids_t3 = input_ids.astype(jnp.int32).T.reshape(seq, 1, bsz)
    emb_t_pad = jnp.pad(embed.T, ((0, 0), (0, vpad - vocab)))
    out_t = _head_tc(ids_t3, emb_t_pad, W, b.reshape(vocab, 1))
    return jnp.transpose(out_t, (2, 0, 1))


# speedup vs baseline: 10.0267x; 1.0007x over previous
import jax
import jax.numpy as jnp
from jax.experimental import pallas as pl


def _head_body(idsT_ref, embT_ref, w_ref, bcol_ref, o_ref):
    vpad = embT_ref.shape[1]
    bsz = idsT_ref.shape[2]
    ids = idsT_ref[0, 0, :]
    iota = jax.lax.broadcasted_iota(jnp.int32, (vpad, bsz), 0)
    onehot_t = (iota == ids[None, :]).astype(jnp.float32)
    h_t = jnp.dot(embT_ref[...], onehot_t, preferred_element_type=jnp.float32)
    o_ref[0] = (
        jnp.dot(w_ref[...], h_t, preferred_element_type=jnp.float32)
        + bcol_ref[...]
    )


def _head_tc(ids_t3, emb_t_pad, w, b_col):
    seq, _, bsz = ids_t3.shape
    vocab, e = w.shape
    vpad = emb_t_pad.shape[1]
    return pl.pallas_call(
        _head_body,
        grid=(seq,),
        in_specs=[
            pl.BlockSpec((1, 1, bsz), lambda i: (i, 0, 0)),
            pl.BlockSpec((e, vpad), lambda i: (0, 0)),
            pl.BlockSpec((vocab, e), lambda i: (0, 0)),
            pl.BlockSpec((vocab, 1), lambda i: (0, 0)),
        ],
        out_specs=pl.BlockSpec((1, vocab, bsz), lambda i: (i, 0, 0)),
        out_shape=jax.ShapeDtypeStruct((seq, vocab, bsz), jnp.float32),
    )(ids_t3, emb_t_pad, w, b_col)


def kernel(input_ids, embed, W, b):
    bsz, seq = input_ids.shape
    vocab = W.shape[0]
    vpad = (vocab + 127) // 128 * 128
    ids_t3 = input_ids.astype(jnp.int32).T.reshape(seq, 1, bsz)
    emb_t_pad = jnp.pad(embed.T, ((0, 0), (0, vpad - vocab)))
    out_t = _head_tc(ids_t3, emb_t_pad, W, b.reshape(vocab, 1))
    return jnp.transpose(out_t, (2, 0, 1))


# speedup vs baseline: 10.2812x; 1.0254x over previous
import jax
import jax.numpy as jnp
from jax.experimental import pallas as pl


def _head_body(idsT_ref, embT_ref, w_ref, o_ref):
    vpad = embT_ref.shape[1]
    bsz = idsT_ref.shape[2]
    ids = idsT_ref[0, 0, :]
    iota = jax.lax.broadcasted_iota(jnp.int32, (vpad, bsz), 0)
    onehot_t = (iota == ids[None, :]).astype(jnp.float32)
    h_aug = jnp.dot(embT_ref[...], onehot_t, preferred_element_type=jnp.float32)
    o_ref[0] = jnp.dot(w_ref[...], h_aug, preferred_element_type=jnp.float32)


def _head_tc(ids_t3, emb_aug, w_aug):
    seq, _, bsz = ids_t3.shape
    vocab, e1 = w_aug.shape
    vpad = emb_aug.shape[1]
    return pl.pallas_call(
        _head_body,
        grid=(seq,),
        in_specs=[
            pl.BlockSpec((1, 1, bsz), lambda i: (i, 0, 0)),
            pl.BlockSpec((e1, vpad), lambda i: (0, 0)),
            pl.BlockSpec((vocab, e1), lambda i: (0, 0)),
        ],
        out_specs=pl.BlockSpec((1, vocab, bsz), lambda i: (i, 0, 0)),
        out_shape=jax.ShapeDtypeStruct((seq, vocab, bsz), jnp.float32),
    )(ids_t3, emb_aug, w_aug)


def kernel(input_ids, embed, W, b):
    bsz, seq = input_ids.shape
    vocab = W.shape[0]
    vpad = (vocab + 127) // 128 * 128
    ids_t3 = input_ids.astype(jnp.int32).T.reshape(seq, 1, bsz)
    emb_t = jnp.pad(embed.T, ((0, 0), (0, vpad - vocab)))
    ones_row = jnp.ones((1, vpad), jnp.float32)
    emb_aug = jnp.concatenate([emb_t, ones_row], axis=0)
    w_aug = jnp.concatenate([W, b.reshape(vocab, 1)], axis=1)
    out_t = _head_tc(ids_t3, emb_aug, w_aug)
    return jnp.transpose(out_t, (2, 0, 1))
